# Initial kernel scaffold; baseline (speedup 1.0000x reference)
import functools

import jax
import jax.numpy as jnp
from jax import lax
from jax.experimental import pallas as pl
from jax.experimental.pallas import tpu as pltpu
from jax.experimental.pallas import tpu_sc as plsc

NC = 2
NS = 16
LANES = 16
NW = NC * NS

IDX_W = 128


@functools.lru_cache(maxsize=None)
def _build(B, L, V):
    CH = 16
    ROWS_W = B // NW
    NCH = ROWS_W // CH
    N = CH * L
    NJ = N // IDX_W
    assert B % NW == 0 and ROWS_W % CH == 0 and N % IDX_W == 0

    inv_l = 1.0 / float(L)
    bias = 0.0 + 0.0001 * float(L) / 10.0 + 0.0001 * float(L) / 10.0

    mesh = plsc.VectorSubcoreMesh(core_axis_name="c", subcore_axis_name="s")

    @functools.partial(
        pl.kernel,
        out_type=jax.ShapeDtypeStruct((B,), jnp.float32),
        mesh=mesh,
        scratch_types=[
            pltpu.VMEM((NJ, IDX_W), jnp.int32),
            pltpu.VMEM((NJ, IDX_W), jnp.int32),
            pltpu.VMEM((N,), jnp.float32),
            pltpu.VMEM((N,), jnp.float32),
            pltpu.VMEM((N,), jnp.float32),
            pltpu.VMEM((N,), jnp.float32),
            pltpu.VMEM((N,), jnp.float32),
            pltpu.VMEM((N,), jnp.float32),
            pltpu.VMEM((LANES,), jnp.float32),
            pltpu.SemaphoreType.DMA,
        ],
    )
    def sc_kernel(xi_hbm, yi_hbm, xv_hbm, yv_hbm, wtf, utf, wti, uti,
                  out_hbm, xi, yi, xv, yv, g0, g1, g2, g3, outv, sem_g):
        cid = lax.axis_index("c")
        sid = lax.axis_index("s")
        wid = sid * NC + cid
        row_lanes = lax.iota(jnp.int32, (LANES,), 0) * L

        for c in range(NCH):
            row0 = wid * (ROWS_W * L // IDX_W) + c * NJ
            pltpu.sync_copy(xi_hbm.at[pl.ds(row0, NJ)], xi)
            pltpu.sync_copy(yi_hbm.at[pl.ds(row0, NJ)], yi)
            base = wid * ROWS_W * L + c * N
            pltpu.sync_copy(xv_hbm.at[pl.ds(base, N)], xv)
            pltpu.sync_copy(yv_hbm.at[pl.ds(base, N)], yv)

            def fire(j, carry):
                dst = pl.ds(j * IDX_W, IDX_W)
                pltpu.async_copy(wtf.at[xi.at[j]], g0.at[dst], sem_g)
                pltpu.async_copy(wti.at[xi.at[j]], g2.at[dst], sem_g)
                pltpu.async_copy(utf.at[yi.at[j]], g1.at[dst], sem_g)
                pltpu.async_copy(uti.at[yi.at[j]], g3.at[dst], sem_g)
                return carry
            lax.fori_loop(0, NJ, fire, 0)
            for gbuf in (g0, g1, g2, g3):
                pltpu.make_async_copy(wtf.at[pl.ds(0, N)], gbuf, sem_g).wait()

            def cbody(j, carry):
                a_tf, a_ti = carry
                idxv = row_lanes + j
                v0 = plsc.load_gather(g0, [idxv])
                v1 = plsc.load_gather(g1, [idxv])
                v2 = plsc.load_gather(g2, [idxv])
                v3 = plsc.load_gather(g3, [idxv])
                xq = plsc.load_gather(xv, [idxv])
                yq = plsc.load_gather(yv, [idxv])
                a_tf = a_tf + (v0 + v1)
                a_ti = a_ti + v2 * xq + v3 * yq
                return (a_tf, a_ti)
            zero = jnp.zeros((LANES,), jnp.float32)
            a_tf, a_ti = lax.fori_loop(0, L, cbody, (zero, zero))

            z = a_tf * inv_l + a_ti + bias
            outv[...] = 1.0 / (1.0 + jnp.exp(-z))
            pltpu.sync_copy(outv, out_hbm.at[pl.ds(wid * ROWS_W + c * CH, CH)])

    return sc_kernel


def kernel(x_idx, y_idx, x_tfidf, y_tfidf, W_tf, U_tf, W_tfidf, U_tfidf):
    B, L = x_idx.shape
    V = W_tf.shape[0]
    xi2 = x_idx.reshape(-1, IDX_W)
    yi2 = y_idx.reshape(-1, IDX_W)
    xvf = x_tfidf.reshape(-1)
    yvf = y_tfidf.reshape(-1)
    return _build(B, L, V)(xi2, yi2, xvf, yvf, W_tf, U_tf, W_tfidf, U_tfidf)


# speedup vs baseline: 1.3585x; 1.3585x over previous
import functools

import jax
import jax.numpy as jnp
from jax import lax
from jax.experimental import pallas as pl
from jax.experimental.pallas import tpu as pltpu
from jax.experimental.pallas import tpu_sc as plsc

NC = 2
NS = 16
LANES = 16
NW = NC * NS

IDX_W = 128


@functools.lru_cache(maxsize=None)
def _build(B, L, V):
    CH = 16
    ROWS_W = B // NW
    NCH = ROWS_W // CH
    N = CH * L
    NJ = N // IDX_W
    assert B % NW == 0 and ROWS_W % CH == 0 and N % IDX_W == 0

    inv_l = 1.0 / float(L)
    bias = 0.0 + 0.0001 * float(L) / 10.0 + 0.0001 * float(L) / 10.0

    mesh = plsc.VectorSubcoreMesh(core_axis_name="c", subcore_axis_name="s")

    @functools.partial(
        pl.kernel,
        out_type=jax.ShapeDtypeStruct((B,), jnp.float32),
        mesh=mesh,
        scratch_types=[
            pltpu.VMEM((N,), jnp.int32),
            pltpu.VMEM((N,), jnp.int32),
            pltpu.VMEM((N,), jnp.float32),
            pltpu.VMEM((N,), jnp.float32),
            pltpu.VMEM((N,), jnp.float32),
            pltpu.VMEM((N,), jnp.float32),
            pltpu.VMEM((N,), jnp.float32),
            pltpu.VMEM((N,), jnp.float32),
            pltpu.VMEM((LANES,), jnp.float32),
            pltpu.SemaphoreType.DMA,
        ],
        compiler_params=pltpu.CompilerParams(needs_layout_passes=False),
    )
    def sc_kernel(xi_hbm, yi_hbm, xv_hbm, yv_hbm, wtf, utf, wti, uti,
                  out_hbm, xi, yi, xv, yv, g0, g1, g2, g3, outv, sem_g):
        cid = lax.axis_index("c")
        sid = lax.axis_index("s")
        wid = sid * NC + cid
        row_lanes = lax.iota(jnp.int32, LANES) * L

        for c in range(NCH):
            base = wid * ROWS_W * L + c * N
            pltpu.sync_copy(xi_hbm.at[pl.ds(base, N)], xi)
            pltpu.sync_copy(yi_hbm.at[pl.ds(base, N)], yi)
            pltpu.sync_copy(xv_hbm.at[pl.ds(base, N)], xv)
            pltpu.sync_copy(yv_hbm.at[pl.ds(base, N)], yv)

            def fire(j, carry):
                win = pl.ds(j * IDX_W, IDX_W)
                pltpu.async_copy(wtf.at[xi.at[win]], g0.at[win], sem_g)
                pltpu.async_copy(wti.at[xi.at[win]], g2.at[win], sem_g)
                pltpu.async_copy(utf.at[yi.at[win]], g1.at[win], sem_g)
                pltpu.async_copy(uti.at[yi.at[win]], g3.at[win], sem_g)
                return carry
            lax.fori_loop(0, NJ, fire, 0)
            for gbuf in (g0, g1, g2, g3):
                pltpu.make_async_copy(wtf.at[pl.ds(0, N)], gbuf, sem_g).wait()

            def cbody(j, carry):
                a_tf, a_ti = carry
                idxv = row_lanes + j
                v0 = plsc.load_gather(g0, [idxv])
                v1 = plsc.load_gather(g1, [idxv])
                v2 = plsc.load_gather(g2, [idxv])
                v3 = plsc.load_gather(g3, [idxv])
                xq = plsc.load_gather(xv, [idxv])
                yq = plsc.load_gather(yv, [idxv])
                a_tf = a_tf + (v0 + v1)
                a_ti = a_ti + v2 * xq + v3 * yq
                return (a_tf, a_ti)
            zero = jnp.zeros((LANES,), jnp.float32)
            a_tf, a_ti = lax.fori_loop(0, L, cbody, (zero, zero))

            z = a_tf * inv_l + a_ti + bias
            outv[...] = 1.0 / (1.0 + jnp.exp(-z))
            pltpu.sync_copy(outv, out_hbm.at[pl.ds(wid * ROWS_W + c * CH, CH)])

    return sc_kernel


def kernel(x_idx, y_idx, x_tfidf, y_tfidf, W_tf, U_tf, W_tfidf, U_tfidf):
    B, L = x_idx.shape
    V = W_tf.shape[0]
    xi2 = x_idx.reshape(-1)
    yi2 = y_idx.reshape(-1)
    xvf = x_tfidf.reshape(-1)
    yvf = y_tfidf.reshape(-1)
    return _build(B, L, V)(xi2, yi2, xvf, yvf, W_tf, U_tf, W_tfidf, U_tfidf)


# speedup vs baseline: 1.5344x; 1.1295x over previous
import functools

import jax
import jax.numpy as jnp
from jax import lax
from jax.experimental import pallas as pl
from jax.experimental.pallas import tpu as pltpu
from jax.experimental.pallas import tpu_sc as plsc

NC = 2
NS = 16
LANES = 16

IDX_W = 128


@functools.lru_cache(maxsize=None)
def _build(B, L, V):
    CH = 16
    ROWS_T = B // NS
    NCH = ROWS_T // CH
    N = CH * L
    NJ = N // IDX_W
    assert B % NS == 0 and ROWS_T % CH == 0 and N % IDX_W == 0 and V % 8 == 0
    VCHUNK = (V // (8 * NS)) * 8
    VTAIL = V - NS * VCHUNK
    NBB = 4
    assert VCHUNK % (8 * NBB) == 0
    BB = VCHUNK // NBB

    inv_l = 1.0 / float(L)

    mesh = plsc.VectorSubcoreMesh(core_axis_name="c", subcore_axis_name="s")
    vm = pltpu.VMEM

    @functools.partial(
        pl.kernel,
        out_type=jax.ShapeDtypeStruct((2 * B,), jnp.float32),
        mesh=mesh,
        scratch_types=[
            pltpu.VMEM_SHARED((V,), jnp.float32),
            vm((N,), jnp.int32), vm((N,), jnp.int32),
            vm((N,), jnp.float32), vm((N,), jnp.float32),
            vm((N,), jnp.float32), vm((N,), jnp.float32),
            vm((N,), jnp.float32), vm((N,), jnp.float32),
            vm((LANES,), jnp.float32),
            vm((BB,), jnp.float32),
            pltpu.SemaphoreType.DMA, pltpu.SemaphoreType.DMA,
        ],
        compiler_params=pltpu.CompilerParams(needs_layout_passes=False),
    )
    def sc_kernel(xi_hbm, yi_hbm, xv_hbm, yv_hbm, wtf, wti, utf, uti,
                  out_hbm, sh_ti,
                  ix0, ix1, vv0, vv1, gtf0, gtf1, gti0, gti1,
                  outv, bb, sem0, sem1):
        ixs, vvs = (ix0, ix1), (vv0, vv1)
        gtfs, gtis, sems = (gtf0, gtf1), (gti0, gti1), (sem0, sem1)

        cid = lax.axis_index("c")
        sid = lax.axis_index("s")
        row_lanes = lax.iota(jnp.int32, LANES) * L

        def stage_tables(t_ti):
            for src, dst in ((t_ti, sh_ti),):
                for k in range(NBB):
                    win = pl.ds(sid * VCHUNK + k * BB, BB)
                    pltpu.sync_copy(src.at[win], bb)
                    pltpu.sync_copy(bb, dst.at[win])
                if VTAIL:
                    tail = pl.ds(NS * VCHUNK, VTAIL)

                    @pl.when(sid == NS - 1)
                    def _():
                        pltpu.sync_copy(src.at[tail], bb.at[pl.ds(0, VTAIL)])
                        pltpu.sync_copy(bb.at[pl.ds(0, VTAIL)], dst.at[tail])

        def pipeline(idx_hbm, val_hbm, t_tf, t_ti, out_base):
            tbase = sid * ROWS_T * L

            def stage(c):
                p = c % 2
                base = tbase + c * N
                pltpu.sync_copy(idx_hbm.at[pl.ds(base, N)], ixs[p])
                pltpu.sync_copy(val_hbm.at[pl.ds(base, N)], vvs[p])

            def fire(c):
                p = c % 2

                def body(j, carry):
                    win = pl.ds(j * IDX_W, IDX_W)
                    pltpu.async_copy(t_tf.at[ixs[p].at[win]], gtfs[p].at[win], sems[p])
                    pltpu.async_copy(t_ti.at[ixs[p].at[win]], gtis[p].at[win], sems[p])
                    return carry
                lax.fori_loop(0, NJ, body, 0)

            def drain(c):
                p = c % 2
                pltpu.make_async_copy(wtf.at[pl.ds(0, N)], gtfs[p], sems[p]).wait()
                pltpu.make_async_copy(wtf.at[pl.ds(0, N)], gtis[p], sems[p]).wait()

            def compute(c):
                p = c % 2

                def cbody(j, carry):
                    a_tf, a_ti = carry
                    idxv = row_lanes + j
                    vtf = plsc.load_gather(gtfs[p], [idxv])
                    vti = plsc.load_gather(gtis[p], [idxv])
                    vq = plsc.load_gather(vvs[p], [idxv])
                    return (a_tf + vtf, a_ti + vti * vq)
                zero = jnp.zeros((LANES,), jnp.float32)
                a_tf, a_ti = lax.fori_loop(0, L, cbody, (zero, zero), unroll=4)

                outv[...] = a_tf * inv_l + a_ti
                dst = pl.ds(out_base + sid * ROWS_T + c * CH, CH)
                pltpu.sync_copy(outv, out_hbm.at[dst])

            stage(0)
            fire(0)
            for c in range(NCH):
                if c + 1 < NCH:
                    stage(c + 1)
                    fire(c + 1)
                drain(c)
                compute(c)

        @pl.when(cid == 0)
        def _():
            stage_tables(wti)

        @pl.when(cid == 1)
        def _():
            stage_tables(uti)

        plsc.subcore_barrier()

        @pl.when(cid == 0)
        def _():
            pipeline(xi_hbm, xv_hbm, wtf, wti, 0)

        @pl.when(cid == 1)
        def _():
            pipeline(yi_hbm, yv_hbm, utf, uti, B)

    bias = 0.0 + 0.0001 * float(L) / 10.0 + 0.0001 * float(L) / 10.0

    def fin_body(p_ref, o_ref):
        z = p_ref[0, :] + p_ref[1, :] + bias
        o_ref[...] = 1.0 / (1.0 + jnp.exp(-z))

    finisher = pl.pallas_call(
        fin_body,
        out_shape=jax.ShapeDtypeStruct((B,), jnp.float32),
    )

    def run(xi, yi, xvf, yvf, W_tf, U_tf, W_tfidf, U_tfidf):
        partials = sc_kernel(xi, yi, xvf, yvf, W_tf, W_tfidf, U_tf, U_tfidf)
        return finisher(partials.reshape(2, B))

    return run


def kernel(x_idx, y_idx, x_tfidf, y_tfidf, W_tf, U_tf, W_tfidf, U_tfidf):
    B, L = x_idx.shape
    V = W_tf.shape[0]
    xi = x_idx.reshape(-1)
    yi = y_idx.reshape(-1)
    xvf = x_tfidf.reshape(-1)
    yvf = y_tfidf.reshape(-1)
    return _build(B, L, V)(xi, yi, xvf, yvf, W_tf, U_tf, W_tfidf, U_tfidf)
